# baseline (speedup 1.0000x reference)
import functools

import jax
import jax.numpy as jnp
from jax import lax
from jax.experimental import pallas as pl
from jax.experimental.pallas import tpu as pltpu
from jax.experimental.pallas import tpu_sc as plsc

_D = 2048
_E = 16
_TOPK = 2
_T = 16384
_BT = 1024

_NW = 32
_TW = _T // _NW
_G = _TW // 16


def _gate_body(h_ref, wt_ref, b_ref, logits_ref):
    h = h_ref[...]
    wt = wt_ref[...]
    logits = jnp.dot(h, wt, preferred_element_type=jnp.float32)
    logits_ref[...] = logits.T + b_ref[...]


def _gate_logits_t(hidden_states, W, b):
    wt = W.T
    b2 = b.reshape(_E, 1)
    return pl.pallas_call(
        _gate_body,
        grid=(_T // _BT,),
        in_specs=[
            pl.BlockSpec((_BT, _D), lambda i: (i, 0)),
            pl.BlockSpec((_D, _E), lambda i: (0, 0)),
            pl.BlockSpec((_E, 1), lambda i: (0, 0)),
        ],
        out_specs=pl.BlockSpec((_E, _BT), lambda i: (0, i)),
        out_shape=jax.ShapeDtypeStruct((_E, _T), jnp.float32),
        compiler_params=pltpu.CompilerParams(
            dimension_semantics=("arbitrary",),
        ),
    )(hidden_states, wt, b2)


def _route_body(lt_hbm, wts_hbm, sel_hbm, mask_hbm,
                lt_v, w_v, s_v, m_v):
    wid = lax.axis_index("s") * 2 + lax.axis_index("c")
    base = wid * _TW
    for e in range(_E):
        pltpu.sync_copy(lt_hbm.at[pl.ds(e * _T + base, _TW)], lt_v.at[e])

    one = jnp.ones((16,), jnp.int32)
    zero = jnp.zeros((16,), jnp.int32)

    def group(g, carry):
        rows = [lt_v[e, pl.ds(g * 16, 16)] for e in range(_E)]
        v1 = rows[0]
        for e in range(1, _E):
            v1 = jnp.maximum(v1, rows[e])
        i1 = jnp.zeros((16,), jnp.int32)
        for e in range(_E - 1, -1, -1):
            i1 = jnp.where(rows[e] == v1, e, i1)
        neg = jnp.full((16,), -jnp.inf, jnp.float32)
        rows2 = [jnp.where(i1 == e, neg, rows[e]) for e in range(_E)]
        v2 = rows2[0]
        for e in range(1, _E):
            v2 = jnp.maximum(v2, rows2[e])
        i2 = jnp.zeros((16,), jnp.int32)
        for e in range(_E - 1, -1, -1):
            i2 = jnp.where(rows2[e] == v2, e, i2)

        e2 = jnp.exp(v2 - v1)
        den = 1.0 + e2
        w1 = 1.0 / den
        w2 = e2 / den

        w_v[0, pl.ds(g * 16, 16)] = w1
        w_v[1, pl.ds(g * 16, 16)] = w2
        s_v[0, pl.ds(g * 16, 16)] = i1
        s_v[1, pl.ds(g * 16, 16)] = i2
        for e in range(_E):
            m_v[2 * e, pl.ds(g * 16, 16)] = jnp.where(i1 == e, one, zero)
            m_v[2 * e + 1, pl.ds(g * 16, 16)] = jnp.where(i2 == e, one, zero)
        return carry

    lax.fori_loop(0, _G, group, 0)

    for k in range(_TOPK):
        pltpu.sync_copy(w_v.at[k], wts_hbm.at[pl.ds(k * _T + base, _TW)])
        pltpu.sync_copy(s_v.at[k], sel_hbm.at[pl.ds(k * _T + base, _TW)])
    for r in range(_E * _TOPK):
        pltpu.sync_copy(m_v.at[r], mask_hbm.at[pl.ds(r * _T + base, _TW)])


_route = functools.partial(
    pl.kernel,
    mesh=plsc.VectorSubcoreMesh(core_axis_name="c", subcore_axis_name="s"),
    out_type=[
        jax.ShapeDtypeStruct((_TOPK * _T,), jnp.float32),
        jax.ShapeDtypeStruct((_TOPK * _T,), jnp.int32),
        jax.ShapeDtypeStruct((_E * _TOPK * _T,), jnp.int32),
    ],
    scratch_types=[
        pltpu.VMEM((_E, _TW), jnp.float32),
        pltpu.VMEM((_TOPK, _TW), jnp.float32),
        pltpu.VMEM((_TOPK, _TW), jnp.int32),
        pltpu.VMEM((_E * _TOPK, _TW), jnp.int32),
    ],
)(_route_body)


def kernel(hidden_states, W, b):
    lt = _gate_logits_t(hidden_states, W, b)
    wts_f, sel_f, mask_f = _route(lt.reshape(_E * _T))
    return (lt.T,
            wts_f.reshape(_TOPK, _T).T,
            sel_f.reshape(_TOPK, _T).T,
            mask_f.reshape(_E, _TOPK, _T))


# speedup vs baseline: 1.1331x; 1.1331x over previous
import functools

import jax
import jax.numpy as jnp
from jax import lax
from jax.experimental import pallas as pl
from jax.experimental.pallas import tpu as pltpu
from jax.experimental.pallas import tpu_sc as plsc

_D = 2048
_E = 16
_TOPK = 2
_T = 16384
_BT = 1024

_NW = 32
_TW = _T // _NW
_G = _TW // 16


def _gate_body(h_ref, wt_ref, b_ref, logits_ref):
    h = h_ref[...]
    wt = wt_ref[...]
    logits = jnp.dot(h, wt, preferred_element_type=jnp.float32)
    logits_ref[...] = logits.T + b_ref[...]


def _gate_logits_t(hidden_states, W, b):
    wt = W.T
    b2 = b.reshape(_E, 1)
    return pl.pallas_call(
        _gate_body,
        grid=(_T // _BT,),
        in_specs=[
            pl.BlockSpec((_BT, _D), lambda i: (i, 0)),
            pl.BlockSpec((_D, _E), lambda i: (0, 0)),
            pl.BlockSpec((_E, 1), lambda i: (0, 0)),
        ],
        out_specs=pl.BlockSpec((_E, _BT), lambda i: (0, i)),
        out_shape=jax.ShapeDtypeStruct((_E, _T), jnp.float32),
        compiler_params=pltpu.CompilerParams(
            dimension_semantics=("arbitrary",),
        ),
    )(hidden_states, wt, b2)


def _route_body(lt_hbm, wts_hbm, sel_hbm, mask_hbm,
                lt_v, w_v, s_v, m_v, sem):
    wid = lax.axis_index("s") * 2 + lax.axis_index("c")
    base = wid * _TW
    in_copies = [
        pltpu.async_copy(lt_hbm.at[pl.ds(e * _T + base, _TW)], lt_v.at[e], sem)
        for e in range(_E)
    ]
    for c in in_copies:
        c.wait()

    one = jnp.ones((16,), jnp.int32)
    zero = jnp.zeros((16,), jnp.int32)

    def group(g, carry):
        rows = [lt_v[e, pl.ds(g * 16, 16)] for e in range(_E)]
        v1 = rows[0]
        for e in range(1, _E):
            v1 = jnp.maximum(v1, rows[e])
        i1 = jnp.zeros((16,), jnp.int32)
        for e in range(_E - 1, -1, -1):
            i1 = jnp.where(rows[e] == v1, e, i1)
        neg = jnp.full((16,), -jnp.inf, jnp.float32)
        rows2 = [jnp.where(i1 == e, neg, rows[e]) for e in range(_E)]
        v2 = rows2[0]
        for e in range(1, _E):
            v2 = jnp.maximum(v2, rows2[e])
        i2 = jnp.zeros((16,), jnp.int32)
        for e in range(_E - 1, -1, -1):
            i2 = jnp.where(rows2[e] == v2, e, i2)

        e2 = jnp.exp(v2 - v1)
        den = 1.0 + e2
        w1 = 1.0 / den
        w2 = e2 / den

        w_v[0, pl.ds(g * 16, 16)] = w1
        w_v[1, pl.ds(g * 16, 16)] = w2
        s_v[0, pl.ds(g * 16, 16)] = i1
        s_v[1, pl.ds(g * 16, 16)] = i2
        for e in range(_E):
            m_v[2 * e, pl.ds(g * 16, 16)] = jnp.where(i1 == e, one, zero)
            m_v[2 * e + 1, pl.ds(g * 16, 16)] = jnp.where(i2 == e, one, zero)
        return carry

    lax.fori_loop(0, _G, group, 0)

    out_copies = []
    for k in range(_TOPK):
        out_copies.append(
            pltpu.async_copy(w_v.at[k], wts_hbm.at[pl.ds(k * _T + base, _TW)], sem))
        out_copies.append(
            pltpu.async_copy(s_v.at[k], sel_hbm.at[pl.ds(k * _T + base, _TW)], sem))
    for r in range(_E * _TOPK):
        out_copies.append(
            pltpu.async_copy(m_v.at[r], mask_hbm.at[pl.ds(r * _T + base, _TW)], sem))
    for c in out_copies:
        c.wait()


_route = functools.partial(
    pl.kernel,
    mesh=plsc.VectorSubcoreMesh(core_axis_name="c", subcore_axis_name="s"),
    out_type=[
        jax.ShapeDtypeStruct((_TOPK * _T,), jnp.float32),
        jax.ShapeDtypeStruct((_TOPK * _T,), jnp.int32),
        jax.ShapeDtypeStruct((_E * _TOPK * _T,), jnp.int32),
    ],
    scratch_types=[
        pltpu.VMEM((_E, _TW), jnp.float32),
        pltpu.VMEM((_TOPK, _TW), jnp.float32),
        pltpu.VMEM((_TOPK, _TW), jnp.int32),
        pltpu.VMEM((_E * _TOPK, _TW), jnp.int32),
        pltpu.SemaphoreType.DMA,
    ],
)(_route_body)


def kernel(hidden_states, W, b):
    lt = _gate_logits_t(hidden_states, W, b)
    wts_f, sel_f, mask_f = _route(lt.reshape(_E * _T))
    return (lt.T,
            wts_f.reshape(_TOPK, _T).T,
            sel_f.reshape(_TOPK, _T).T,
            mask_f.reshape(_E, _TOPK, _T))


# speedup vs baseline: 1.6604x; 1.4654x over previous
import jax
import jax.numpy as jnp
from jax import lax
from jax.experimental import pallas as pl
from jax.experimental.pallas import tpu as pltpu

_D = 2048
_E = 16
_TOPK = 2
_T = 16384
_BT = 1024


def _route_block(logits, b_col, logits_ref, wts_ref, sel_ref, mask_ref, col0):
    lt = logits.T + b_col
    logits_ref[:, pl.ds(col0, _BT)] = lt

    e_iota = lax.broadcasted_iota(jnp.int32, (_E, _BT), 0)
    v1 = jnp.max(lt, axis=0, keepdims=True)
    i1 = jnp.min(jnp.where(lt == v1, e_iota, _E), axis=0, keepdims=True)
    l2 = jnp.where(e_iota == i1, jnp.float32(-jnp.inf), lt)
    v2 = jnp.max(l2, axis=0, keepdims=True)
    i2 = jnp.min(jnp.where(l2 == v2, e_iota, _E), axis=0, keepdims=True)

    e2 = jnp.exp(v2 - v1)
    denom = 1.0 + e2
    wts_ref[:, pl.ds(col0, _BT)] = jnp.concatenate(
        [1.0 / denom, e2 / denom], axis=0)
    sel_ref[:, pl.ds(col0, _BT)] = jnp.concatenate([i1, i2], axis=0)

    r_iota = lax.broadcasted_iota(jnp.int32, (_E * _TOPK, _BT), 0)
    sel_r = jnp.where((r_iota & 1) == 0, i1, i2)
    mask_ref[:, pl.ds(col0, _BT)] = (sel_r == (r_iota >> 1)).astype(jnp.int32)


def _router_body(ha_ref, hb_ref, wt_ref, b_ref,
                 logits_ref, wts_ref, sel_ref, mask_ref):
    wt = wt_ref[...]
    b_col = b_ref[...]
    la = jnp.dot(ha_ref[...], wt, preferred_element_type=jnp.float32)
    _route_block(la, b_col, logits_ref, wts_ref, sel_ref, mask_ref, 0)
    lb = jnp.dot(hb_ref[...], wt, preferred_element_type=jnp.float32)
    _route_block(lb, b_col, logits_ref, wts_ref, sel_ref, mask_ref, _BT)


def kernel(hidden_states, W, b):
    wt = W.T
    b2 = b.reshape(_E, 1)
    grid = (_T // (2 * _BT),)
    logits_t, wts_t, sel_t, mask_t = pl.pallas_call(
        _router_body,
        grid=grid,
        in_specs=[
            pl.BlockSpec((_BT, _D), lambda i: (2 * i, 0)),
            pl.BlockSpec((_BT, _D), lambda i: (2 * i + 1, 0)),
            pl.BlockSpec((_D, _E), lambda i: (0, 0)),
            pl.BlockSpec((_E, 1), lambda i: (0, 0)),
        ],
        out_specs=[
            pl.BlockSpec((_E, 2 * _BT), lambda i: (0, i)),
            pl.BlockSpec((_TOPK, 2 * _BT), lambda i: (0, i)),
            pl.BlockSpec((_TOPK, 2 * _BT), lambda i: (0, i)),
            pl.BlockSpec((_E * _TOPK, 2 * _BT), lambda i: (0, i)),
        ],
        out_shape=[
            jax.ShapeDtypeStruct((_E, _T), jnp.float32),
            jax.ShapeDtypeStruct((_TOPK, _T), jnp.float32),
            jax.ShapeDtypeStruct((_TOPK, _T), jnp.int32),
            jax.ShapeDtypeStruct((_E * _TOPK, _T), jnp.int32),
        ],
        compiler_params=pltpu.CompilerParams(
            dimension_semantics=("arbitrary",),
        ),
    )(hidden_states, hidden_states, wt, b2)
    return (logits_t.T, wts_t.T, sel_t.T, mask_t.reshape(_E, _TOPK, _T))


# speedup vs baseline: 1.7174x; 1.0343x over previous
import jax
import jax.numpy as jnp
from jax import lax
from jax.experimental import pallas as pl
from jax.experimental.pallas import tpu as pltpu

_D = 2048
_E = 16
_TOPK = 2
_T = 16384
_BT = 1024


def _router_body(h_ref, wt_ref, b_ref, logits_ref, wts_ref, sel_ref, mask_ref):
    h = h_ref[...]
    wt = wt_ref[...]
    logits = jnp.dot(h, wt, preferred_element_type=jnp.float32)
    lt = logits.T + b_ref[...]
    logits_ref[...] = lt

    e_iota = lax.broadcasted_iota(jnp.int32, (_E, _BT), 0)
    v1 = jnp.max(lt, axis=0, keepdims=True)
    i1 = jnp.min(jnp.where(lt == v1, e_iota, _E), axis=0, keepdims=True)
    l2 = jnp.where(e_iota == i1, jnp.float32(-jnp.inf), lt)
    v2 = jnp.max(l2, axis=0, keepdims=True)
    i2 = jnp.min(jnp.where(l2 == v2, e_iota, _E), axis=0, keepdims=True)

    e2 = jnp.exp(v2 - v1)
    denom = 1.0 + e2
    wts_ref[...] = jnp.concatenate([1.0 / denom, e2 / denom], axis=0)
    sel_ref[...] = jnp.concatenate([i1, i2], axis=0)

    r_iota = lax.broadcasted_iota(jnp.int32, (_E * _TOPK, _BT), 0)
    sel_r = jnp.where((r_iota & 1) == 0, i1, i2)
    mask_ref[...] = (sel_r == (r_iota >> 1)).astype(jnp.int32)


def kernel(hidden_states, W, b):
    wt = W.T
    b2 = b.reshape(_E, 1)
    grid = (_T // _BT,)
    logits_t, wts_t, sel_t, mask_t = pl.pallas_call(
        _router_body,
        grid=grid,
        in_specs=[
            pl.BlockSpec((_BT, _D), lambda i: (i, 0)),
            pl.BlockSpec((_D, _E), lambda i: (0, 0)),
            pl.BlockSpec((_E, 1), lambda i: (0, 0)),
        ],
        out_specs=[
            pl.BlockSpec((_E, _BT), lambda i: (0, i)),
            pl.BlockSpec((_TOPK, _BT), lambda i: (0, i)),
            pl.BlockSpec((_TOPK, _BT), lambda i: (0, i)),
            pl.BlockSpec((_E * _TOPK, _BT), lambda i: (0, i)),
        ],
        out_shape=[
            jax.ShapeDtypeStruct((_E, _T), jnp.float32),
            jax.ShapeDtypeStruct((_TOPK, _T), jnp.float32),
            jax.ShapeDtypeStruct((_TOPK, _T), jnp.int32),
            jax.ShapeDtypeStruct((_E * _TOPK, _T), jnp.int32),
        ],
        compiler_params=pltpu.CompilerParams(
            dimension_semantics=("parallel",),
        ),
    )(hidden_states, wt, b2)
    return (logits_t.T, wts_t.T, sel_t.T, mask_t.reshape(_E, _TOPK, _T))
